# baseline (speedup 1.0000x reference)
import jax
import jax.numpy as jnp
from jax import lax
from jax.experimental import pallas as pl
from jax.experimental.pallas import tpu as pltpu

B, T, IDIM = 8, 2048, 80
HDIM, CDIM, TNUM = 160, 16, 10
NLOOP = HDIM // CDIM
K2_MEAN = sum((k + 1) ** 2 for k in range(NLOOP)) / NLOOP
K1_MEAN = sum((k + 1) for k in range(NLOOP)) / NLOOP

HB = B // 2


def _one_seq(x, thr, w1, b1, w2, b2):
    h = jnp.tanh(
        lax.dot_general(x, w1, (((1,), (0,)), ((), ())),
                        preferred_element_type=jnp.float32)
        + b1
    )
    e = (
        lax.dot_general(h, w2, (((1,), (0,)), ((), ())),
                        preferred_element_type=jnp.float32)
        + b2
    )

    def s(a, i):
        return pltpu.roll(a, T - i, 0)

    u = s(x, 1) + s(x, 2)
    w = u + s(u, 2)
    w = w + s(w, 4)
    msum = w + s(u, 8)

    t_idx = lax.broadcasted_iota(jnp.int32, (T, 1), 0)
    vmask = (t_idx < thr).astype(jnp.float32)

    q = e * vmask
    pm = msum * vmask

    def gram(a_mat, b_mat):
        return lax.dot_general(a_mat, b_mat, (((0,), (0,)), ((), ())),
                               preferred_element_type=jnp.float32)

    g_combined = (K2_MEAN * gram(q, e)
                  - (2.0 * K1_MEAN / TNUM) * gram(q, msum)
                  + (1.0 / (TNUM * TNUM)) * gram(pm, msum))
    ii = lax.broadcasted_iota(jnp.int32, (IDIM, IDIM), 0)
    jj = lax.broadcasted_iota(jnp.int32, (IDIM, IDIM), 1)
    eye = (ii == jj).astype(jnp.float32)
    return jnp.sum(g_combined * eye)


def _loss_kernel(ilens_ref, x0_ref, x1_ref, w1_ref, b1_ref, w2_ref, b2_ref,
                 out_ref):
    g = pl.program_id(0)
    w1 = w1_ref[...]
    b1 = b1_ref[...]
    w2 = w2_ref[...]
    b2 = b2_ref[...]
    part0 = _one_seq(x0_ref[0], ilens_ref[g] - TNUM, w1, b1, w2, b2)
    part1 = _one_seq(x1_ref[0], ilens_ref[g + HB] - TNUM, w1, b1, w2, b2)

    @pl.when(g == 0)
    def _():
        out_ref[0, 0] = 0.0

    out_ref[0, 0] += part0 + part1


@jax.jit
def _run(xs_pad, ilens, W1, b1, W2, b2):
    grid_spec = pltpu.PrefetchScalarGridSpec(
        num_scalar_prefetch=1,
        grid=(HB,),
        in_specs=[
            pl.BlockSpec((1, T, IDIM), lambda g, ilens: (g, 0, 0)),
            pl.BlockSpec((1, T, IDIM), lambda g, ilens: (g + HB, 0, 0)),
            pl.BlockSpec((IDIM, HDIM), lambda g, ilens: (0, 0)),
            pl.BlockSpec((1, HDIM), lambda g, ilens: (0, 0)),
            pl.BlockSpec((HDIM, IDIM), lambda g, ilens: (0, 0)),
            pl.BlockSpec((1, IDIM), lambda g, ilens: (0, 0)),
        ],
        out_specs=pl.BlockSpec(memory_space=pltpu.SMEM),
    )
    out = pl.pallas_call(
        _loss_kernel,
        grid_spec=grid_spec,
        out_shape=jax.ShapeDtypeStruct((1, 1), jnp.float32),
    )(ilens.astype(jnp.int32), xs_pad, xs_pad,
      W1, b1.reshape(1, HDIM), W2, b2.reshape(1, IDIM))
    return out[0, 0]


def kernel(xs_pad, ilens, ys_pad, W1, b1, W2, b2):
    del ys_pad
    return _run(xs_pad, ilens, W1, b1, W2, b2)


# speedup vs baseline: 1.0009x; 1.0009x over previous
import jax
import jax.numpy as jnp
from jax import lax
from jax.experimental import pallas as pl
from jax.experimental.pallas import tpu as pltpu

B, T, IDIM = 8, 2048, 80
HDIM, CDIM, TNUM = 160, 16, 10
NLOOP = HDIM // CDIM
K2_MEAN = sum((k + 1) ** 2 for k in range(NLOOP)) / NLOOP
K1_MEAN = sum((k + 1) for k in range(NLOOP)) / NLOOP


def _one_seq(x, thr, w1, b1, w2, b2):
    h = jnp.tanh(
        lax.dot_general(x, w1, (((1,), (0,)), ((), ())),
                        preferred_element_type=jnp.float32)
        + b1
    )
    e = (
        lax.dot_general(h, w2, (((1,), (0,)), ((), ())),
                        preferred_element_type=jnp.float32)
        + b2
    )

    def s(a, i):
        return jnp.concatenate([a[i:], a[:i]], axis=0)

    u = s(x, 1) + s(x, 2)
    w = u + s(u, 2)
    w = w + s(w, 4)
    msum = w + s(u, 8)

    t_idx = lax.broadcasted_iota(jnp.int32, (T, 1), 0)
    vmask = (t_idx < thr).astype(jnp.float32)

    q = e * vmask
    pm = msum * vmask
    a_part = jnp.sum(q * e)
    b_part = jnp.sum(q * msum)
    c_part = jnp.sum(pm * msum)
    return (K2_MEAN * a_part
            - (2.0 * K1_MEAN / TNUM) * b_part
            + (1.0 / (TNUM * TNUM)) * c_part)


def _loss_kernel(ilens_ref, x_hbm_ref, w1_ref, b1_ref, w2_ref, b2_ref,
                 out_ref, xbuf, sems):
    g = pl.program_id(0)

    @pl.when(g == 0)
    def _():
        for i in range(B):
            pltpu.make_async_copy(
                x_hbm_ref.at[i], xbuf.at[i], sems.at[i]
            ).start()
        out_ref[0, 0] = 0.0

    pltpu.make_async_copy(x_hbm_ref.at[g], xbuf.at[g], sems.at[g]).wait()
    part = _one_seq(xbuf[g], ilens_ref[g] - TNUM,
                    w1_ref[...], b1_ref[...], w2_ref[...], b2_ref[...])
    out_ref[0, 0] += part


@jax.jit
def _run(xs_pad, ilens, W1, b1, W2, b2):
    grid_spec = pltpu.PrefetchScalarGridSpec(
        num_scalar_prefetch=1,
        grid=(B,),
        in_specs=[
            pl.BlockSpec(memory_space=pltpu.MemorySpace.HBM),
            pl.BlockSpec((IDIM, HDIM), lambda g, ilens: (0, 0)),
            pl.BlockSpec((1, HDIM), lambda g, ilens: (0, 0)),
            pl.BlockSpec((HDIM, IDIM), lambda g, ilens: (0, 0)),
            pl.BlockSpec((1, IDIM), lambda g, ilens: (0, 0)),
        ],
        out_specs=pl.BlockSpec(memory_space=pltpu.SMEM),
        scratch_shapes=[
            pltpu.VMEM((B, T, IDIM), jnp.float32),
            pltpu.SemaphoreType.DMA((B,)),
        ],
    )
    out = pl.pallas_call(
        _loss_kernel,
        grid_spec=grid_spec,
        out_shape=jax.ShapeDtypeStruct((1, 1), jnp.float32),
    )(ilens.astype(jnp.int32), xs_pad,
      W1, b1.reshape(1, HDIM), W2, b2.reshape(1, IDIM))
    return out[0, 0]


def kernel(xs_pad, ilens, ys_pad, W1, b1, W2, b2):
    del ys_pad
    return _run(xs_pad, ilens, W1, b1, W2, b2)


# speedup vs baseline: 1.4350x; 1.4337x over previous
import jax
import jax.numpy as jnp
from jax import lax
from jax.experimental import pallas as pl
from jax.experimental.pallas import tpu as pltpu

B, T, IDIM = 8, 2048, 80
HDIM, CDIM, TNUM = 160, 16, 10
NLOOP = HDIM // CDIM
K2_MEAN = sum((k + 1) ** 2 for k in range(NLOOP)) / NLOOP
K1_MEAN = sum((k + 1) for k in range(NLOOP)) / NLOOP


def _loss_kernel(ilens_ref, x_ref, w1_ref, b1_ref, w2_ref, b2_ref, out_ref):
    g = pl.program_id(0)
    x = x_ref[0]

    h = jnp.tanh(
        lax.dot_general(w1_ref[...], x, (((0,), (0,)), ((), ())),
                        preferred_element_type=jnp.float32)
        + b1_ref[...]
    )
    e = (
        lax.dot_general(w2_ref[...], h, (((0,), (0,)), ((), ())),
                        preferred_element_type=jnp.float32)
        + b2_ref[...]
    )

    def s(a, i):
        return jnp.concatenate([a[:, i:], a[:, :i]], axis=1)

    u = s(x, 1) + s(x, 2)
    w = u + s(u, 2)
    w = w + s(w, 4)
    msum = w + s(u, 8)

    t_idx = lax.broadcasted_iota(jnp.int32, (IDIM, T), 1)
    vmask = (t_idx < (ilens_ref[g] - TNUM)).astype(jnp.float32)

    q = e * vmask
    pm = msum * vmask
    a_part = jnp.sum(q * e)
    b_part = jnp.sum(q * msum)
    c_part = jnp.sum(pm * msum)
    part = (K2_MEAN * a_part
            - (2.0 * K1_MEAN / TNUM) * b_part
            + (1.0 / (TNUM * TNUM)) * c_part)

    @pl.when(g == 0)
    def _():
        out_ref[0, 0] = 0.0

    out_ref[0, 0] += part


@jax.jit
def _run(xs_t, ilens, W1, b1, W2, b2):
    grid_spec = pltpu.PrefetchScalarGridSpec(
        num_scalar_prefetch=1,
        grid=(B,),
        in_specs=[
            pl.BlockSpec((1, IDIM, T), lambda g, ilens: (g, 0, 0)),
            pl.BlockSpec((IDIM, HDIM), lambda g, ilens: (0, 0)),
            pl.BlockSpec((HDIM, 1), lambda g, ilens: (0, 0)),
            pl.BlockSpec((HDIM, IDIM), lambda g, ilens: (0, 0)),
            pl.BlockSpec((IDIM, 1), lambda g, ilens: (0, 0)),
        ],
        out_specs=pl.BlockSpec(memory_space=pltpu.SMEM),
    )
    out = pl.pallas_call(
        _loss_kernel,
        grid_spec=grid_spec,
        out_shape=jax.ShapeDtypeStruct((1, 1), jnp.float32),
    )(ilens.astype(jnp.int32), xs_t,
      W1, b1.reshape(HDIM, 1), W2, b2.reshape(IDIM, 1))
    return out[0, 0]


def kernel(xs_pad, ilens, ys_pad, W1, b1, W2, b2):
    del ys_pad
    xs_t = jnp.transpose(xs_pad, (0, 2, 1))
    return _run(xs_t, ilens, W1, b1, W2, b2)
